# baseline (speedup 1.0000x reference)
import functools

import jax
import jax.numpy as jnp
from jax import lax
from jax.experimental import pallas as pl
from jax.experimental.pallas import tpu as pltpu
from jax.experimental.pallas import tpu_sc as plsc

N = 10000
NPAD = 10240
D = 128
E = 320000
NC = 2
NS = 16
L = 16
NW = NC * NS
BLK = 128
NBLK = E // BLK
NBLKP = 2560
EPAD = NBLKP * BLK
NJ = NBLKP // NW
ROWS_PER_SUB = NPAD // NS
ZR = 128
SEG = 1024
_GRID = NPAD // SEG


@functools.cache
def _mesh():
    return plsc.VectorSubcoreMesh(core_axis_name="c", subcore_axis_name="s",
                                  num_cores=NC, num_subcores=NS)


def _edge_loop(body):
    def wrapped(j, _):
        body(j)
        return 0
    lax.fori_loop(0, NJ, wrapped, 0)


def _zero_acc(zeros_hbm, zb, acc, s):
    pltpu.sync_copy(zeros_hbm, zb)
    for j in range(ROWS_PER_SUB // ZR):
        pltpu.sync_copy(zb, acc.at[pl.ds(s * ROWS_PER_SUB + j * ZR, ZR)])


def _write_partial(out_hbm, zb, acc, c, s):
    for j in range(ROWS_PER_SUB // ZR):
        pltpu.sync_copy(acc.at[pl.ds(s * ROWS_PER_SUB + j * ZR, ZR)], zb)
        pltpu.sync_copy(
            zb, out_hbm.at[pl.ds(c * NPAD + s * ROWS_PER_SUB + j * ZR, ZR)])


@functools.cache
def _sc_count_fn():
    return functools.partial(
        pl.kernel,
        out_type=jax.ShapeDtypeStruct((NC * NPAD, D), jnp.float32),
        mesh=_mesh(),
        scratch_types=[
            pltpu.VMEM((BLK,), jnp.int32),
            pltpu.VMEM((BLK, D), jnp.float32),
            pltpu.VMEM((ZR, D), jnp.float32),
            pltpu.VMEM_SHARED((NPAD, D), jnp.float32),
        ],
    )(_sc_count)


def _sc_count(zeros_hbm, ones_hbm, dst_hbm, out_hbm, didx, rows, zb, acc):
    c = lax.axis_index("c")
    s = lax.axis_index("s")
    w = s * NC + c

    _zero_acc(zeros_hbm, zb, acc, s)
    pltpu.sync_copy(ones_hbm, rows)
    plsc.subcore_barrier()

    def blk(j):
        base = (w + j * NW) * BLK
        pltpu.sync_copy(dst_hbm.at[pl.ds(base, BLK)], didx)
        pltpu.sync_copy(rows, acc.at[didx], add=True)
    _edge_loop(blk)
    plsc.subcore_barrier()
    _write_partial(out_hbm, zb, acc, c, s)


@functools.cache
def _sc_aggregate_fn():
    return functools.partial(
        pl.kernel,
        out_type=jax.ShapeDtypeStruct((NC * NPAD, D), jnp.float32),
        mesh=_mesh(),
        scratch_types=[
            pltpu.VMEM((BLK,), jnp.int32),
            pltpu.VMEM((BLK,), jnp.int32),
            pltpu.VMEM((BLK, D), jnp.float32),
            pltpu.VMEM((BLK,), jnp.int32),
            pltpu.VMEM((BLK,), jnp.int32),
            pltpu.VMEM((BLK, D), jnp.float32),
            pltpu.VMEM_SHARED((NPAD, D), jnp.float32),
            pltpu.SemaphoreType.DMA,
            pltpu.SemaphoreType.DMA,
        ],
    )(_sc_aggregate)


def _sc_aggregate(zeros_hbm, src_hbm, dst_hbm, hp_hbm, out_hbm,
                  sidx0, didx0, rows0, sidx1, didx1, rows1,
                  acc, sem0, sem1):
    c = lax.axis_index("c")
    s = lax.axis_index("s")
    w = s * NC + c
    sidx = (sidx0, sidx1)
    didx = (didx0, didx1)
    rows = (rows0, rows1)
    sem = (sem0, sem1)

    _zero_acc(zeros_hbm, rows0, acc, s)
    plsc.subcore_barrier()

    def fetch(b, bid):
        base = bid * BLK
        pltpu.sync_copy(src_hbm.at[pl.ds(base, BLK)], sidx[b])
        pltpu.sync_copy(dst_hbm.at[pl.ds(base, BLK)], didx[b])
        pltpu.async_copy(hp_hbm.at[sidx[b]], rows[b], sem[b])

    fetch(0, w)
    fetch(1, w + NW)
    def pair(jj, _):
        for b in range(2):
            j = 2 * jj + b
            pltpu.make_async_copy(hp_hbm.at[sidx[b]], rows[b], sem[b]).wait()
            pltpu.sync_copy(rows[b], acc.at[didx[b]], add=True)
            nxt = lax.rem(w + (j + 2) * NW, jnp.int32(NBLKP))
            fetch(b, nxt)
        return 0
    lax.fori_loop(0, NJ // 2, pair, 0)
    for b in range(2):
        pltpu.make_async_copy(hp_hbm.at[sidx[b]], rows[b], sem[b]).wait()
    plsc.subcore_barrier()
    _write_partial(out_hbm, rows0, acc, c, s)


def _tc1_body(x_ref, w_ref, degp_ref, hp_ref, dis_ref):
    deg = degp_ref[0, :, 0:1] + degp_ref[1, :, 0:1] + 1.0
    dis = lax.rsqrt(deg)
    h = jnp.dot(x_ref[...], w_ref[...], preferred_element_type=jnp.float32)
    hp_ref[...] = h * dis
    dis_ref[...] = jnp.broadcast_to(dis, (SEG, D))


def _tc2_body(p_ref, hp1_ref, dis_ref, b_ref, w_ref, hp2_ref):
    dis = dis_ref[:, 0:1]
    agg = p_ref[0] + p_ref[1] + hp1_ref[...]
    h2 = jnp.maximum(agg * dis + b_ref[...], 0.0)
    hp2_ref[...] = jnp.dot(h2, w_ref[...],
                           preferred_element_type=jnp.float32) * dis


def _tc3_body(q_ref, hp2_ref, dis_ref, b_ref, out_ref):
    dis = dis_ref[:, 0:1]
    agg = q_ref[0] + q_ref[1] + hp2_ref[...]
    out_ref[...] = agg * dis + b_ref[...]


_ROW2 = pl.BlockSpec((SEG, D), lambda i: (i, 0))
_ROW3 = pl.BlockSpec((NC, SEG, D), lambda i: (0, i, 0))
_W_SPEC = pl.BlockSpec((D, D), lambda i: (0, 0))
_B_SPEC = pl.BlockSpec((1, D), lambda i: (0, 0))
_SDS = jax.ShapeDtypeStruct((NPAD, D), jnp.float32)

_tc1 = pl.pallas_call(
    _tc1_body, grid=(_GRID,),
    in_specs=[_ROW2, _W_SPEC, _ROW3],
    out_specs=[_ROW2, _ROW2], out_shape=[_SDS, _SDS])

_tc2 = pl.pallas_call(
    _tc2_body, grid=(_GRID,),
    in_specs=[_ROW3, _ROW2, _ROW2, _B_SPEC, _W_SPEC],
    out_specs=_ROW2, out_shape=_SDS)

_tc3 = pl.pallas_call(
    _tc3_body, grid=(_GRID,),
    in_specs=[_ROW3, _ROW2, _ROW2, _B_SPEC],
    out_specs=_ROW2, out_shape=_SDS)


def kernel(x, edge_index, W1, b1, W2, b2):
    src = edge_index[0].astype(jnp.int32)
    dst = edge_index[1].astype(jnp.int32)
    b1 = b1.reshape(1, D).astype(jnp.float32)
    b2 = b2.reshape(1, D).astype(jnp.float32)
    xp = jnp.zeros((NPAD, D), jnp.float32).at[:N].set(x)
    zrows = jnp.zeros((ZR, D), jnp.float32)
    orows = jnp.ones((BLK, D), jnp.float32)

    sc_count = _sc_count_fn()
    sc_aggregate = _sc_aggregate_fn()

    padn = EPAD - E
    srcp = jnp.concatenate([src, jnp.zeros((padn,), jnp.int32)])
    dstp = jnp.concatenate(
        [dst, N + (jnp.arange(padn, dtype=jnp.int32) % (NPAD - N))])

    degp = sc_count(zrows, orows, dstp).reshape(NC, NPAD, D)
    hp1, disb = _tc1(xp, W1, degp)
    p = sc_aggregate(zrows, srcp, dstp, hp1).reshape(NC, NPAD, D)
    hp2 = _tc2(p, hp1, disb, b1, W2)
    q = sc_aggregate(zrows, srcp, dstp, hp2).reshape(NC, NPAD, D)
    return _tc3(q, hp2, disb, b2)[:N]


# speedup vs baseline: 1.7526x; 1.7526x over previous
import functools

import jax
import jax.numpy as jnp
from jax import lax
from jax.experimental import pallas as pl
from jax.experimental.pallas import tpu as pltpu
from jax.experimental.pallas import tpu_sc as plsc

N = 10000
NPAD = 10240
D = 128
E = 320000
NC = 2
NS = 16
L = 16
NW = NC * NS
BLK = 128
NBLK = E // BLK
ROWS_PER_SUB = NPAD // NS
ZR = 128
SEG = 1024
_GRID = NPAD // SEG


@functools.cache
def _mesh():
    return plsc.VectorSubcoreMesh(core_axis_name="c", subcore_axis_name="s",
                                  num_cores=NC, num_subcores=NS)


def _edge_loop(w, body):
    nj = jnp.int32(NBLK // NW) + (w < NBLK % NW).astype(jnp.int32)
    def wrapped(j, _):
        body(j)
        return 0
    lax.fori_loop(0, nj, wrapped, 0)


def _zero_acc(zeros_hbm, zb, acc, s):
    pltpu.sync_copy(zeros_hbm, zb)
    for j in range(ROWS_PER_SUB // ZR):
        pltpu.sync_copy(zb, acc.at[pl.ds(s * ROWS_PER_SUB + j * ZR, ZR)])


def _write_partial(out_hbm, zb, acc, c, s):
    for j in range(ROWS_PER_SUB // ZR):
        pltpu.sync_copy(acc.at[pl.ds(s * ROWS_PER_SUB + j * ZR, ZR)], zb)
        pltpu.sync_copy(
            zb, out_hbm.at[pl.ds(c * NPAD + s * ROWS_PER_SUB + j * ZR, ZR)])


@functools.cache
def _sc_count_fn():
    return functools.partial(
        pl.kernel,
        out_type=jax.ShapeDtypeStruct((NC * NPAD, D), jnp.float32),
        mesh=_mesh(),
        scratch_types=[
            pltpu.VMEM((BLK,), jnp.int32),
            pltpu.VMEM((BLK, D), jnp.float32),
            pltpu.VMEM((ZR, D), jnp.float32),
            pltpu.VMEM_SHARED((NPAD, D), jnp.float32),
        ],
    )(_sc_count)


def _sc_count(zeros_hbm, ones_hbm, dst_hbm, out_hbm, didx, rows, zb, acc):
    c = lax.axis_index("c")
    s = lax.axis_index("s")
    w = s * NC + c

    _zero_acc(zeros_hbm, zb, acc, s)
    pltpu.sync_copy(ones_hbm, rows)
    plsc.subcore_barrier()

    def blk(j):
        base = (w + j * NW) * BLK
        pltpu.sync_copy(dst_hbm.at[pl.ds(base, BLK)], didx)
        pltpu.sync_copy(rows, acc.at[didx], add=True)
    _edge_loop(w, blk)
    plsc.subcore_barrier()
    _write_partial(out_hbm, zb, acc, c, s)


@functools.cache
def _sc_aggregate_fn():
    return functools.partial(
        pl.kernel,
        out_type=jax.ShapeDtypeStruct((NC * NPAD, D), jnp.float32),
        mesh=_mesh(),
        scratch_types=[
            pltpu.VMEM((2, BLK), jnp.int32),
            pltpu.VMEM((BLK, D), jnp.float32),
            pltpu.VMEM((ZR, D), jnp.float32),
            pltpu.VMEM_SHARED((NPAD, D), jnp.float32),
            pltpu.SemaphoreType.DMA,
        ],
    )(_sc_aggregate)


def _sc_aggregate(zeros_hbm, sd_hbm, hp_hbm, out_hbm,
                  sd, rows, zb, acc, sem):
    c = lax.axis_index("c")
    s = lax.axis_index("s")
    w = s * NC + c

    _zero_acc(zeros_hbm, zb, acc, s)
    plsc.subcore_barrier()

    def blk(j):
        bid = w + j * NW
        pltpu.sync_copy(sd_hbm.at[pl.ds(2 * bid, 2)], sd)
        pltpu.async_copy(hp_hbm.at[sd.at[0]], rows, sem).wait()
        pltpu.sync_copy(rows, acc.at[sd.at[1]], add=True)
    _edge_loop(w, blk)
    plsc.subcore_barrier()
    _write_partial(out_hbm, zb, acc, c, s)


def _tc1_body(x_ref, w_ref, degp_ref, hp_ref, dis_ref):
    deg = degp_ref[0, :, 0:1] + degp_ref[1, :, 0:1] + 1.0
    dis = lax.rsqrt(deg)
    h = jnp.dot(x_ref[...], w_ref[...], preferred_element_type=jnp.float32)
    hp_ref[...] = h * dis
    dis_ref[...] = jnp.broadcast_to(dis, (SEG, D))


def _tc2_body(p_ref, hp1_ref, dis_ref, b_ref, w_ref, hp2_ref):
    dis = dis_ref[:, 0:1]
    agg = p_ref[0] + p_ref[1] + hp1_ref[...]
    h2 = jnp.maximum(agg * dis + b_ref[...], 0.0)
    hp2_ref[...] = jnp.dot(h2, w_ref[...],
                           preferred_element_type=jnp.float32) * dis


def _tc3_body(q_ref, hp2_ref, dis_ref, b_ref, out_ref):
    dis = dis_ref[:, 0:1]
    agg = q_ref[0] + q_ref[1] + hp2_ref[...]
    out_ref[...] = agg * dis + b_ref[...]


_ROW2 = pl.BlockSpec((SEG, D), lambda i: (i, 0))
_ROW3 = pl.BlockSpec((NC, SEG, D), lambda i: (0, i, 0))
_W_SPEC = pl.BlockSpec((D, D), lambda i: (0, 0))
_B_SPEC = pl.BlockSpec((1, D), lambda i: (0, 0))
_SDS = jax.ShapeDtypeStruct((NPAD, D), jnp.float32)

_tc1 = pl.pallas_call(
    _tc1_body, grid=(_GRID,),
    in_specs=[_ROW2, _W_SPEC, _ROW3],
    out_specs=[_ROW2, _ROW2], out_shape=[_SDS, _SDS])

_tc2 = pl.pallas_call(
    _tc2_body, grid=(_GRID,),
    in_specs=[_ROW3, _ROW2, _ROW2, _B_SPEC, _W_SPEC],
    out_specs=_ROW2, out_shape=_SDS)

_tc3 = pl.pallas_call(
    _tc3_body, grid=(_GRID,),
    in_specs=[_ROW3, _ROW2, _ROW2, _B_SPEC],
    out_specs=_ROW2, out_shape=_SDS)


def kernel(x, edge_index, W1, b1, W2, b2):
    src = edge_index[0].astype(jnp.int32)
    dst = edge_index[1].astype(jnp.int32)
    b1 = b1.reshape(1, D).astype(jnp.float32)
    b2 = b2.reshape(1, D).astype(jnp.float32)
    xp = jnp.zeros((NPAD, D), jnp.float32).at[:N].set(x)
    zrows = jnp.zeros((ZR, D), jnp.float32)
    orows = jnp.ones((BLK, D), jnp.float32)

    sc_count = _sc_count_fn()
    sc_aggregate = _sc_aggregate_fn()

    sd = jnp.stack([src.reshape(NBLK, BLK), dst.reshape(NBLK, BLK)],
                   axis=1).reshape(2 * NBLK, BLK)

    degp = sc_count(zrows, orows, dst).reshape(NC, NPAD, D)
    hp1, disb = _tc1(xp, W1, degp)
    p = sc_aggregate(zrows, sd, hp1).reshape(NC, NPAD, D)
    hp2 = _tc2(p, hp1, disb, b1, W2)
    q = sc_aggregate(zrows, sd, hp2).reshape(NC, NPAD, D)
    return _tc3(q, hp2, disb, b2)[:N]
